# baseline (speedup 1.0000x reference)
import math

import jax
import jax.numpy as jnp
from jax.experimental import pallas as pl
from jax.experimental.pallas import tpu as pltpu

B, S, HS = 1, 2048, 2048
H, KVH, DH = 16, 4, 128
GROUPS = H // KVH
THETA = 10000.0

BSR = 512
BSQ = 256
NQ = S // BSQ
WQKV = (H + 2 * KVH) * DH


def _rot(x):
    return jnp.concatenate([-x[:, DH // 2:], x[:, :DH // 2]], axis=1)


def _qkv_body(x_ref, wq_ref, wk_ref, wv_ref, cos_ref, sin_ref,
              cosq_ref, sinq_ref, q_ref, k_ref, v_ref, w_s):
    @pl.when(pl.program_id(0) == 0)
    def _():
        w_s[:, :H * DH] = wq_ref[...].astype(jnp.bfloat16)
        w_s[:, H * DH:(H + KVH) * DH] = wk_ref[...].astype(jnp.bfloat16)
        w_s[:, (H + KVH) * DH:] = wv_ref[...].astype(jnp.bfloat16)

    x = x_ref[...].astype(jnp.bfloat16)
    qkv = jnp.dot(x, w_s[...], preferred_element_type=jnp.float32)
    cos = cos_ref[...]
    sin = sin_ref[...]
    cosq = cosq_ref[...]
    sinq = sinq_ref[...]
    for h in range(H):
        qh = qkv[:, h * DH:(h + 1) * DH]
        q_ref[h] = (qh * cosq + _rot(qh) * sinq).astype(jnp.bfloat16)
    off = H * DH
    for g in range(KVH):
        kg = qkv[:, off + g * DH: off + (g + 1) * DH]
        k_ref[g] = (kg * cos + _rot(kg) * sin).astype(jnp.bfloat16)
    off2 = (H + KVH) * DH
    for g in range(KVH):
        v_ref[g] = qkv[:, off2 + g * DH: off2 + (g + 1) * DH].astype(jnp.bfloat16)


def _make_attn_body(i_base):
    def _attn_body(q_ref, k_ref, v_ref, o_ref):
        i = pl.program_id(1)
        k = k_ref[0]
        v = v_ref[0]
        width = k.shape[0]
        row = (i_base + i) * BSQ + jax.lax.broadcasted_iota(
            jnp.int32, (BSQ, width), 0)
        col = jax.lax.broadcasted_iota(jnp.int32, (BSQ, width), 1)
        mask = col <= row
        for hh in range(GROUPS):
            q = q_ref[hh]
            s = jax.lax.dot_general(q, k, (((1,), (1,)), ((), ())),
                                    preferred_element_type=jnp.float32)
            s = jnp.where(mask, s, -1e30)
            m = jnp.max(s, axis=1, keepdims=True)
            p = jnp.exp((s - m).astype(jnp.bfloat16))
            l = jnp.sum(p, axis=1, keepdims=True, dtype=jnp.float32)
            acc = jnp.dot(p, v, preferred_element_type=jnp.float32)
            o_ref[:, hh * DH:(hh + 1) * DH] = (acc / l).astype(jnp.bfloat16)
    return _attn_body


def _proj_body(o_ref, w_ref, out_ref, w_s):
    @pl.when(pl.program_id(0) == 0)
    def _():
        w_s[...] = w_ref[...].astype(jnp.bfloat16)
    out_ref[...] = jnp.dot(o_ref[...].astype(jnp.bfloat16), w_s[...],
                           preferred_element_type=jnp.float32)


def kernel(hidden_states, position_ids, Wq, Wk, Wv, Wo):
    x = hidden_states[0]
    pos = position_ids[0].astype(jnp.float32)
    inv_freq = 1.0 / (THETA ** (jnp.arange(0, DH, 2, dtype=jnp.float32) / DH))
    freqs = pos[:, None] * inv_freq[None, :]
    emb = jnp.concatenate([freqs, freqs], axis=1)
    cos = jnp.cos(emb)
    sin = jnp.sin(emb)
    scale = 1.0 / math.sqrt(DH)
    cosq = cos * scale
    sinq = sin * scale

    q, k, v = pl.pallas_call(
        _qkv_body,
        grid=(S // BSR,),
        in_specs=[
            pl.BlockSpec((BSR, HS), lambda i: (i, 0)),
            pl.BlockSpec((HS, H * DH), lambda i: (0, 0)),
            pl.BlockSpec((HS, KVH * DH), lambda i: (0, 0)),
            pl.BlockSpec((HS, KVH * DH), lambda i: (0, 0)),
            pl.BlockSpec((BSR, DH), lambda i: (i, 0)),
            pl.BlockSpec((BSR, DH), lambda i: (i, 0)),
            pl.BlockSpec((BSR, DH), lambda i: (i, 0)),
            pl.BlockSpec((BSR, DH), lambda i: (i, 0)),
        ],
        out_specs=[
            pl.BlockSpec((H, BSR, DH), lambda i: (0, i, 0)),
            pl.BlockSpec((KVH, BSR, DH), lambda i: (0, i, 0)),
            pl.BlockSpec((KVH, BSR, DH), lambda i: (0, i, 0)),
        ],
        out_shape=[
            jax.ShapeDtypeStruct((H, S, DH), jnp.bfloat16),
            jax.ShapeDtypeStruct((KVH, S, DH), jnp.bfloat16),
            jax.ShapeDtypeStruct((KVH, S, DH), jnp.bfloat16),
        ],
        scratch_shapes=[pltpu.VMEM((HS, WQKV), jnp.bfloat16)],
    )(x, Wq, Wk, Wv, cos, sin, cosq, sinq)

    o_parts = []
    for c in range(NQ // 2):
        width = (c + 1) * 2 * BSQ
        o_parts.append(pl.pallas_call(
            _make_attn_body(2 * c),
            grid=(KVH, 2),
            in_specs=[
                pl.BlockSpec((GROUPS, BSQ, DH),
                             lambda p, i, c=c: (p, 2 * c + i, 0)),
                pl.BlockSpec((1, width, DH), lambda p, i: (p, 0, 0)),
                pl.BlockSpec((1, width, DH), lambda p, i: (p, 0, 0)),
            ],
            out_specs=pl.BlockSpec((BSQ, GROUPS * DH), lambda p, i: (i, p)),
            out_shape=jax.ShapeDtypeStruct((2 * BSQ, H * DH), jnp.bfloat16),
            compiler_params=pltpu.CompilerParams(
                dimension_semantics=("parallel", "parallel")),
        )(q, k, v))
    o = jnp.concatenate(o_parts, axis=0)

    out = pl.pallas_call(
        _proj_body,
        grid=(S // BSR,),
        in_specs=[
            pl.BlockSpec((BSR, H * DH), lambda i: (i, 0)),
            pl.BlockSpec((H * DH, HS), lambda i: (0, 0)),
        ],
        out_specs=pl.BlockSpec((BSR, HS), lambda i: (i, 0)),
        out_shape=jax.ShapeDtypeStruct((S, HS), jnp.float32),
        scratch_shapes=[pltpu.VMEM((H * DH, HS), jnp.bfloat16)],
    )(o, Wo)
    return out[None]


# speedup vs baseline: 1.1683x; 1.1683x over previous
import math

import jax
import jax.numpy as jnp
from jax.experimental import pallas as pl
from jax.experimental.pallas import tpu as pltpu

B, S, HS = 1, 2048, 2048
H, KVH, DH = 16, 4, 128
GROUPS = H // KVH
THETA = 10000.0

BSR = 512
BSQ = 512
NQ = S // BSQ
WQKV = (H + 2 * KVH) * DH


def _rot(x):
    return jnp.concatenate([-x[:, DH // 2:], x[:, :DH // 2]], axis=1)


def _qkv_body(x_ref, wq_ref, wk_ref, wv_ref, cos_ref, sin_ref,
              cosq_ref, sinq_ref, q_ref, k_ref, v_ref, w_s):
    @pl.when(pl.program_id(0) == 0)
    def _():
        w_s[:, :H * DH] = wq_ref[...].astype(jnp.bfloat16)
        w_s[:, H * DH:(H + KVH) * DH] = wk_ref[...].astype(jnp.bfloat16)
        w_s[:, (H + KVH) * DH:] = wv_ref[...].astype(jnp.bfloat16)

    x = x_ref[...].astype(jnp.bfloat16)
    qkv = jnp.dot(x, w_s[...], preferred_element_type=jnp.float32)
    cos = cos_ref[...]
    sin = sin_ref[...]
    cosq = cosq_ref[...]
    sinq = sinq_ref[...]
    for h in range(H):
        qh = qkv[:, h * DH:(h + 1) * DH]
        q_ref[h] = (qh * cosq + _rot(qh) * sinq).astype(jnp.bfloat16)
    off = H * DH
    for g in range(KVH):
        kg = qkv[:, off + g * DH: off + (g + 1) * DH]
        k_ref[g] = (kg * cos + _rot(kg) * sin).astype(jnp.bfloat16)
    off2 = (H + KVH) * DH
    for g in range(KVH):
        v_ref[g] = qkv[:, off2 + g * DH: off2 + (g + 1) * DH].astype(jnp.bfloat16)


def _make_attn_body(i_base):
    def _attn_body(q_ref, k_ref, v_ref, o_ref):
        i = pl.program_id(1)
        k = k_ref[0]
        v = v_ref[0]
        width = k.shape[0]
        row = (i_base + i) * BSQ + jax.lax.broadcasted_iota(
            jnp.int32, (BSQ, width), 0)
        col = jax.lax.broadcasted_iota(jnp.int32, (BSQ, width), 1)
        mask = col <= row
        for hh in range(GROUPS):
            q = q_ref[hh]
            s = jax.lax.dot_general(q, k, (((1,), (1,)), ((), ())),
                                    preferred_element_type=jnp.float32)
            s = jnp.where(mask, s, -1e30)
            m = jnp.max(s, axis=1, keepdims=True)
            p = jnp.exp(s - m)
            l = jnp.sum(p, axis=1, keepdims=True)
            acc = jnp.dot(p.astype(jnp.bfloat16), v,
                          preferred_element_type=jnp.float32)
            o_ref[:, hh * DH:(hh + 1) * DH] = (acc / l).astype(jnp.bfloat16)
    return _attn_body


def _proj_body(o_ref, w_ref, out_ref, w_s):
    @pl.when(pl.program_id(0) == 0)
    def _():
        w_s[...] = w_ref[...].astype(jnp.bfloat16)
    out_ref[...] = jnp.dot(o_ref[...].astype(jnp.bfloat16), w_s[...],
                           preferred_element_type=jnp.float32)


def kernel(hidden_states, position_ids, Wq, Wk, Wv, Wo):
    x = hidden_states[0]
    pos = position_ids[0].astype(jnp.float32)
    inv_freq = 1.0 / (THETA ** (jnp.arange(0, DH, 2, dtype=jnp.float32) / DH))
    freqs = pos[:, None] * inv_freq[None, :]
    emb = jnp.concatenate([freqs, freqs], axis=1)
    cos = jnp.cos(emb)
    sin = jnp.sin(emb)
    scale = 1.0 / math.sqrt(DH)
    cosq = cos * scale
    sinq = sin * scale

    q, k, v = pl.pallas_call(
        _qkv_body,
        grid=(S // BSR,),
        in_specs=[
            pl.BlockSpec((BSR, HS), lambda i: (i, 0)),
            pl.BlockSpec((HS, H * DH), lambda i: (0, 0)),
            pl.BlockSpec((HS, KVH * DH), lambda i: (0, 0)),
            pl.BlockSpec((HS, KVH * DH), lambda i: (0, 0)),
            pl.BlockSpec((BSR, DH), lambda i: (i, 0)),
            pl.BlockSpec((BSR, DH), lambda i: (i, 0)),
            pl.BlockSpec((BSR, DH), lambda i: (i, 0)),
            pl.BlockSpec((BSR, DH), lambda i: (i, 0)),
        ],
        out_specs=[
            pl.BlockSpec((H, BSR, DH), lambda i: (0, i, 0)),
            pl.BlockSpec((KVH, BSR, DH), lambda i: (0, i, 0)),
            pl.BlockSpec((KVH, BSR, DH), lambda i: (0, i, 0)),
        ],
        out_shape=[
            jax.ShapeDtypeStruct((H, S, DH), jnp.bfloat16),
            jax.ShapeDtypeStruct((KVH, S, DH), jnp.bfloat16),
            jax.ShapeDtypeStruct((KVH, S, DH), jnp.bfloat16),
        ],
        scratch_shapes=[pltpu.VMEM((HS, WQKV), jnp.bfloat16)],
    )(x, Wq, Wk, Wv, cos, sin, cosq, sinq)

    o_parts = []
    for c in range(NQ):
        width = (c + 1) * BSQ
        o_parts.append(pl.pallas_call(
            _make_attn_body(c),
            grid=(KVH, 1),
            in_specs=[
                pl.BlockSpec((GROUPS, BSQ, DH),
                             lambda p, i, c=c: (p, c, 0)),
                pl.BlockSpec((1, width, DH), lambda p, i: (p, 0, 0)),
                pl.BlockSpec((1, width, DH), lambda p, i: (p, 0, 0)),
            ],
            out_specs=pl.BlockSpec((BSQ, GROUPS * DH), lambda p, i: (i, p)),
            out_shape=jax.ShapeDtypeStruct((BSQ, H * DH), jnp.bfloat16),
            compiler_params=pltpu.CompilerParams(
                dimension_semantics=("parallel", "parallel")),
        )(q, k, v))
    o = jnp.concatenate(o_parts, axis=0)

    out = pl.pallas_call(
        _proj_body,
        grid=(S // BSR,),
        in_specs=[
            pl.BlockSpec((BSR, H * DH), lambda i: (i, 0)),
            pl.BlockSpec((H * DH, HS), lambda i: (0, 0)),
        ],
        out_specs=pl.BlockSpec((BSR, HS), lambda i: (i, 0)),
        out_shape=jax.ShapeDtypeStruct((S, HS), jnp.float32),
        scratch_shapes=[pltpu.VMEM((H * DH, HS), jnp.bfloat16)],
    )(o, Wo)
    return out[None]


# speedup vs baseline: 1.2425x; 1.0635x over previous
import math

import jax
import jax.numpy as jnp
from jax.experimental import pallas as pl
from jax.experimental.pallas import tpu as pltpu

B, S, HS = 1, 2048, 2048
H, KVH, DH = 16, 4, 128
GROUPS = H // KVH
THETA = 10000.0

BSR = 512
BSQ = 512
NQ = S // BSQ
WQKV = (H + 2 * KVH) * DH


def _rot(x):
    return jnp.concatenate([-x[:, DH // 2:], x[:, :DH // 2]], axis=1)


def _qkv_body(x_ref, wq_ref, wk_ref, wv_ref, pos_ref, q_ref, k_ref, v_ref, w_s):
    @pl.when(pl.program_id(0) == 0)
    def _():
        w_s[:, :H * DH] = wq_ref[...].astype(jnp.bfloat16)
        w_s[:, H * DH:(H + KVH) * DH] = wk_ref[...].astype(jnp.bfloat16)
        w_s[:, (H + KVH) * DH:] = wv_ref[...].astype(jnp.bfloat16)

    x = x_ref[...].astype(jnp.bfloat16)
    qkv = jnp.dot(x, w_s[...], preferred_element_type=jnp.float32)
    e = (jax.lax.broadcasted_iota(jnp.int32, (1, DH), 1) % (DH // 2))
    inv_freq = jnp.exp(e.astype(jnp.float32) * (-2.0 / DH * math.log(THETA)))
    emb = pos_ref[...] * inv_freq
    cos = jnp.cos(emb)
    sin = jnp.sin(emb)
    scale = 1.0 / math.sqrt(DH)
    cosq = cos * scale
    sinq = sin * scale
    for h in range(H):
        qh = qkv[:, h * DH:(h + 1) * DH]
        q_ref[h] = (qh * cosq + _rot(qh) * sinq).astype(jnp.bfloat16)
    off = H * DH
    for g in range(KVH):
        kg = qkv[:, off + g * DH: off + (g + 1) * DH]
        k_ref[g] = (kg * cos + _rot(kg) * sin).astype(jnp.bfloat16)
    off2 = (H + KVH) * DH
    for g in range(KVH):
        v_ref[g] = qkv[:, off2 + g * DH: off2 + (g + 1) * DH].astype(jnp.bfloat16)


def _make_attn_body(i_base, aliased):
    def _attn_body(*refs):
        if aliased:
            q_ref, k_ref, v_ref, o_ref = refs[1:]
        else:
            q_ref, k_ref, v_ref, o_ref = refs
        i = pl.program_id(1)
        k = k_ref[0]
        v = v_ref[0]
        width = k.shape[0]
        row = (i_base + i) * BSQ + jax.lax.broadcasted_iota(
            jnp.int32, (BSQ, width), 0)
        col = jax.lax.broadcasted_iota(jnp.int32, (BSQ, width), 1)
        mask = col <= row
        for hh in range(GROUPS):
            q = q_ref[hh]
            s = jax.lax.dot_general(q, k, (((1,), (1,)), ((), ())),
                                    preferred_element_type=jnp.float32)
            s = jnp.where(mask, s, -1e30)
            m = jnp.max(s, axis=1, keepdims=True)
            p = jnp.exp(s - m)
            l = jnp.sum(p, axis=1, keepdims=True)
            acc = jnp.dot(p.astype(jnp.bfloat16), v,
                          preferred_element_type=jnp.float32)
            o_ref[:, hh * DH:(hh + 1) * DH] = (acc / l).astype(jnp.bfloat16)
    return _attn_body


def _proj_body(o_ref, w_ref, out_ref, w_s):
    @pl.when(pl.program_id(0) == 0)
    def _():
        w_s[...] = w_ref[...].astype(jnp.bfloat16)
    out_ref[...] = jnp.dot(o_ref[...].astype(jnp.bfloat16), w_s[...],
                           preferred_element_type=jnp.float32)


def kernel(hidden_states, position_ids, Wq, Wk, Wv, Wo):
    x = hidden_states[0]
    pos_col = position_ids.reshape(S, 1).astype(jnp.float32)

    q, k, v = pl.pallas_call(
        _qkv_body,
        grid=(S // BSR,),
        in_specs=[
            pl.BlockSpec((BSR, HS), lambda i: (i, 0)),
            pl.BlockSpec((HS, H * DH), lambda i: (0, 0)),
            pl.BlockSpec((HS, KVH * DH), lambda i: (0, 0)),
            pl.BlockSpec((HS, KVH * DH), lambda i: (0, 0)),
            pl.BlockSpec((BSR, 1), lambda i: (i, 0)),
        ],
        out_specs=[
            pl.BlockSpec((H, BSR, DH), lambda i: (0, i, 0)),
            pl.BlockSpec((KVH, BSR, DH), lambda i: (0, i, 0)),
            pl.BlockSpec((KVH, BSR, DH), lambda i: (0, i, 0)),
        ],
        out_shape=[
            jax.ShapeDtypeStruct((H, S, DH), jnp.bfloat16),
            jax.ShapeDtypeStruct((KVH, S, DH), jnp.bfloat16),
            jax.ShapeDtypeStruct((KVH, S, DH), jnp.bfloat16),
        ],
        scratch_shapes=[pltpu.VMEM((HS, WQKV), jnp.bfloat16)],
    )(x, Wq, Wk, Wv, pos_col)

    o = None
    for c in range(NQ):
        width = (c + 1) * BSQ
        aliased = c > 0
        in_specs = [
            pl.BlockSpec((GROUPS, BSQ, DH), lambda p, i, c=c: (p, c, 0)),
            pl.BlockSpec((1, width, DH), lambda p, i: (p, 0, 0)),
            pl.BlockSpec((1, width, DH), lambda p, i: (p, 0, 0)),
        ]
        args = (q, k, v)
        kwargs = {}
        if aliased:
            in_specs = [pl.BlockSpec(memory_space=pl.ANY)] + in_specs
            args = (o,) + args
            kwargs["input_output_aliases"] = {0: 0}
        o = pl.pallas_call(
            _make_attn_body(c, aliased),
            grid=(KVH, 1),
            in_specs=in_specs,
            out_specs=pl.BlockSpec((BSQ, GROUPS * DH), lambda p, i, c=c: (c, p)),
            out_shape=jax.ShapeDtypeStruct((S, H * DH), jnp.bfloat16),
            compiler_params=pltpu.CompilerParams(
                dimension_semantics=("parallel", "parallel")),
            **kwargs,
        )(*args)

    out = pl.pallas_call(
        _proj_body,
        grid=(S // BSR,),
        in_specs=[
            pl.BlockSpec((BSR, H * DH), lambda i: (i, 0)),
            pl.BlockSpec((H * DH, HS), lambda i: (0, 0)),
        ],
        out_specs=pl.BlockSpec((BSR, HS), lambda i: (i, 0)),
        out_shape=jax.ShapeDtypeStruct((S, HS), jnp.float32),
        scratch_shapes=[pltpu.VMEM((H * DH, HS), jnp.bfloat16)],
    )(o, Wo)
    return out[None]
